# Initial kernel scaffold; baseline (speedup 1.0000x reference)
import jax
import jax.numpy as jnp
from jax.experimental import pallas as pl


def kernel(x, edge_index, Wf, bf, W1, b1, W2, b2):
    raise NotImplementedError("write your pallas kernel here")


# speedup vs baseline: 25.0840x; 25.0840x over previous
import jax
import jax.numpy as jnp
from jax import lax
from jax.experimental import pallas as pl
from jax.experimental.pallas import tpu as pltpu
from jax.experimental.pallas import tpu_sc as plsc

N = 10000
E = 320000
C = 128
NC, NS = 2, 16
NW = NC * NS
CH = 128
NCHUNK = E // CH
RB = 1000
SB = 2000
VL = 16


def _mesh():
    return plsc.VectorSubcoreMesh(core_axis_name="c", subcore_axis_name="s")


_SC_PARAMS = pltpu.CompilerParams(needs_layout_passes=False)


def _reduce_hists(hist, hist_sh, tmp, acc, out_hbm, c, s):
    pltpu.sync_copy(hist, hist_sh.at[pl.ds(s * N, N)])
    plsc.subcore_barrier()

    @pl.when(s < N // SB)
    def _():
        off = s * SB
        pltpu.sync_copy(hist_sh.at[pl.ds(off, SB)], acc)

        @pl.loop(1, NS)
        def _(k):
            pltpu.sync_copy(hist_sh.at[pl.ds(k * N + off, SB)], tmp)

            @pl.loop(0, SB // VL)
            def _(v):
                sl = pl.ds(v * VL, VL)
                acc[sl] = acc[sl] + tmp[sl]

        pltpu.sync_copy(acc, out_hbm.at[pl.ds(c * N + off, SB)])


def _deg_body(dst_hbm, zeros_hbm, deg_out, idx_v, hist, tmp, acc, hist_sh):
    c = lax.axis_index("c")
    s = lax.axis_index("s")
    wid = s * NC + c

    pltpu.sync_copy(zeros_hbm, hist)
    ones16 = jnp.full((VL,), 1.0, jnp.float32)

    @pl.loop(wid, NCHUNK, step=NW)
    def _(ci):
        pltpu.sync_copy(dst_hbm.at[pl.ds(ci * CH, CH)], idx_v)
        for j in range(CH // VL):
            iv = idx_v[pl.ds(j * VL, VL)]
            plsc.addupdate_scatter(hist, [iv], ones16)

    _reduce_hists(hist, hist_sh, tmp, acc, deg_out, c, s)


def _degrees(dst_idx, zeros_n):
    return pl.kernel(
        _deg_body,
        out_type=jax.ShapeDtypeStruct((NC * N,), jnp.float32),
        mesh=_mesh(),
        compiler_params=_SC_PARAMS,
        scratch_types=[
            pltpu.VMEM((CH,), jnp.int32),
            pltpu.VMEM((N,), jnp.float32),
            pltpu.VMEM((SB,), jnp.float32),
            pltpu.VMEM((SB,), jnp.float32),
            pltpu.VMEM_SHARED((NS * N,), jnp.float32),
        ],
    )(dst_idx, zeros_n)


def _mm_body(x_ref, w_ref, dis_ref, y_ref):
    xw = jnp.dot(x_ref[...], w_ref[...], preferred_element_type=jnp.float32)
    y_ref[...] = xw * dis_ref[...]


def _scaled_xw(x, w1, dis2):
    grid = 10
    blk = N // grid
    return pl.pallas_call(
        _mm_body,
        grid=(grid,),
        in_specs=[
            pl.BlockSpec((blk, C), lambda i: (i, 0)),
            pl.BlockSpec((C, C), lambda i: (0, 0)),
            pl.BlockSpec((blk, 1), lambda i: (i, 0)),
        ],
        out_specs=pl.BlockSpec((blk, C), lambda i: (i, 0)),
        out_shape=jax.ShapeDtypeStruct((N, C), jnp.float32),
    )(x, w1, dis2)


def _agg_body(src_hbm, dst_hbm, y_hbm, dis_hbm, zbig_hbm, zeros_hbm,
              agg_out, s_out,
              isrc, idst, rows, dis_v, s_hist, tmp, acc, agg_sh, s_sh, sem):
    c = lax.axis_index("c")
    s = lax.axis_index("s")
    wid = s * NC + c

    @pl.when(s < N // RB)
    def _():
        pltpu.sync_copy(zbig_hbm.at[pl.ds(s * RB, RB)],
                        agg_sh.at[pl.ds(s * RB, RB)])

    pltpu.sync_copy(dis_hbm, dis_v)
    pltpu.sync_copy(zeros_hbm, s_hist)
    plsc.subcore_barrier()

    @pl.loop(wid, NCHUNK, step=NW)
    def _(ci):
        base = ci * CH
        pltpu.sync_copy(src_hbm.at[pl.ds(base, CH)], isrc)
        pltpu.sync_copy(dst_hbm.at[pl.ds(base, CH)], idst)
        gat = pltpu.async_copy(y_hbm.at[isrc], rows, sem)
        for j in range(CH // VL):
            sl = pl.ds(j * VL, VL)
            dv = plsc.load_gather(dis_v, [idst[sl]])
            plsc.addupdate_scatter(s_hist, [isrc[sl]], dv)
        gat.wait()
        pltpu.sync_copy(rows, agg_sh.at[idst], add=True)

    plsc.subcore_barrier()

    @pl.when(s < N // RB)
    def _():
        pltpu.sync_copy(agg_sh.at[pl.ds(s * RB, RB)],
                        agg_out.at[c, pl.ds(s * RB, RB)])

    _reduce_hists(s_hist, s_sh, tmp, acc, s_out, c, s)


def _aggregate(src_idx, dst_idx, y, dis1, zbig, zeros_n):
    return pl.kernel(
        _agg_body,
        out_type=(
            jax.ShapeDtypeStruct((NC, N, C), jnp.float32),
            jax.ShapeDtypeStruct((NC * N,), jnp.float32),
        ),
        mesh=_mesh(),
        compiler_params=_SC_PARAMS,
        scratch_types=[
            pltpu.VMEM((CH,), jnp.int32),
            pltpu.VMEM((CH,), jnp.int32),
            pltpu.VMEM((CH, C), jnp.float32),
            pltpu.VMEM((N,), jnp.float32),
            pltpu.VMEM((N,), jnp.float32),
            pltpu.VMEM((SB,), jnp.float32),
            pltpu.VMEM((SB,), jnp.float32),
            pltpu.VMEM_SHARED((N, C), jnp.float32),
            pltpu.VMEM_SHARED((NS * N,), jnp.float32),
            pltpu.SemaphoreType.DMA,
        ],
    )(src_idx, dst_idx, y, dis1, zbig, zeros_n)


def _final_body(agg0_ref, agg1_ref, y_ref, dis_ref, s0_ref, s1_ref,
                b1_ref, w2_ref, b2_ref, out_ref, vacc):
    i = pl.program_id(0)
    ng = pl.num_programs(0)

    @pl.when(i == 0)
    def _():
        vacc[...] = jnp.zeros_like(vacc)

    dis = dis_ref[...]
    a = agg0_ref[...] + agg1_ref[...] + y_ref[...]
    h = jnp.maximum(a * dis + b1_ref[...], 0.0)
    cw = dis * (s0_ref[...] + s1_ref[...]) + dis * dis
    vacc[...] += jnp.sum(cw * h, axis=0, keepdims=True)

    @pl.when(i == ng - 1)
    def _():
        t = jnp.dot(vacc[...] * (1.0 / N), w2_ref[...],
                    preferred_element_type=jnp.float32) + b2_ref[...]
        m = jnp.max(t, axis=1, keepdims=True)
        lse = jnp.log(jnp.sum(jnp.exp(t - m), axis=1, keepdims=True)) + m
        out_ref[...] = t - lse


def _head(agg0, agg1, y, dis2, s0, s1, b1r, w2, b2r):
    grid = 10
    blk = N // grid
    return pl.pallas_call(
        _final_body,
        grid=(grid,),
        in_specs=[
            pl.BlockSpec((blk, C), lambda i: (i, 0)),
            pl.BlockSpec((blk, C), lambda i: (i, 0)),
            pl.BlockSpec((blk, C), lambda i: (i, 0)),
            pl.BlockSpec((blk, 1), lambda i: (i, 0)),
            pl.BlockSpec((blk, 1), lambda i: (i, 0)),
            pl.BlockSpec((blk, 1), lambda i: (i, 0)),
            pl.BlockSpec((1, C), lambda i: (0, 0)),
            pl.BlockSpec((C, C), lambda i: (0, 0)),
            pl.BlockSpec((1, C), lambda i: (0, 0)),
        ],
        out_specs=pl.BlockSpec((1, C), lambda i: (0, 0)),
        out_shape=jax.ShapeDtypeStruct((1, C), jnp.float32),
        scratch_shapes=[pltpu.VMEM((1, C), jnp.float32)],
    )(agg0, agg1, y, dis2, s0, s1, b1r, w2, b2r)


@jax.jit
def kernel(x, edge_index, Wf, bf, W1, b1, W2, b2):
    del Wf, bf
    src_idx = edge_index[0]
    dst_idx = edge_index[1]

    zbig = jnp.zeros((N, C), jnp.float32)
    zeros_n = jnp.zeros((N,), jnp.float32)

    deg_flat = _degrees(dst_idx, zeros_n)
    dis1 = lax.rsqrt(deg_flat[:N] + deg_flat[N:] + 1.0)
    dis2 = dis1.reshape(N, 1)

    y = _scaled_xw(x, W1, dis2)

    agg2, s_flat = _aggregate(src_idx, dst_idx, y, dis1, zbig, zeros_n)

    return _head(agg2[0], agg2[1], y, dis2,
                 s_flat[:N].reshape(N, 1), s_flat[N:].reshape(N, 1),
                 b1.reshape(1, C), W2, b2.reshape(1, C))


# speedup vs baseline: 36.0594x; 1.4375x over previous
import jax
import jax.numpy as jnp
from jax import lax
from jax.experimental import pallas as pl
from jax.experimental.pallas import tpu as pltpu
from jax.experimental.pallas import tpu_sc as plsc

N = 10000
E = 320000
C = 128
NC, NS = 2, 16
NW = NC * NS
CH = 64
NCHUNK = E // CH
RB = 1000
SB = 1000
VL = 16
PIECE = 2000
NPIECE = E // NW // PIECE


def _mesh():
    return plsc.VectorSubcoreMesh(core_axis_name="c", subcore_axis_name="s")


_SC_PARAMS = pltpu.CompilerParams(needs_layout_passes=False)


def _reduce_hists(hist, hist_sh, tmp, acc, out_hbm, c, s):
    pltpu.sync_copy(hist, hist_sh.at[pl.ds(s * N, N)])
    plsc.subcore_barrier()

    @pl.when(s < N // SB)
    def _():
        off = s * SB
        pltpu.sync_copy(hist_sh.at[pl.ds(off, SB)], acc)

        @pl.loop(1, NS)
        def _(k):
            pltpu.sync_copy(hist_sh.at[pl.ds(k * N + off, SB)], tmp)

            @pl.loop(0, SB // VL)
            def _(v):
                sl = pl.ds(v * VL, VL)
                acc[sl] = acc[sl] + tmp[sl]

        pltpu.sync_copy(acc, out_hbm.at[pl.ds(c * N + off, SB)])


def _deg_body(dst_hbm, zeros_hbm, deg_out, piece0, piece1, hist, tmp, acc,
              hist_sh, semp0, semp1):
    c = lax.axis_index("c")
    s = lax.axis_index("s")
    wid = s * NC + c
    base_w = wid * (E // NW)
    piece = (piece0, piece1)
    semp = (semp0, semp1)

    pltpu.sync_copy(zeros_hbm, hist)
    ones16 = jnp.full((VL,), 1.0, jnp.float32)

    def piece_src(p):
        return dst_hbm.at[pl.ds(base_w + p * PIECE, PIECE)]

    pltpu.async_copy(piece_src(0), piece[0], semp[0])
    for p in range(NPIECE):
        b = p % 2
        if p + 1 < NPIECE:
            pltpu.async_copy(piece_src(p + 1), piece[1 - b], semp[1 - b])
        pltpu.make_async_copy(piece_src(p), piece[b], semp[b]).wait()

        @pl.loop(0, PIECE // VL)
        def _(v):
            iv = piece[b][pl.ds(v * VL, VL)]
            plsc.addupdate_scatter(hist, [iv], ones16)

    _reduce_hists(hist, hist_sh, tmp, acc, deg_out, c, s)


def _degrees(dst_idx, zeros_n):
    return pl.kernel(
        _deg_body,
        out_type=jax.ShapeDtypeStruct((NC * N,), jnp.float32),
        mesh=_mesh(),
        compiler_params=_SC_PARAMS,
        scratch_types=[
            pltpu.VMEM((PIECE,), jnp.int32),
            pltpu.VMEM((PIECE,), jnp.int32),
            pltpu.VMEM((N,), jnp.float32),
            pltpu.VMEM((SB,), jnp.float32),
            pltpu.VMEM((SB,), jnp.float32),
            pltpu.VMEM_SHARED((NS * N,), jnp.float32),
            pltpu.SemaphoreType.DMA,
            pltpu.SemaphoreType.DMA,
        ],
    )(dst_idx, zeros_n)


def _mm_body(x_ref, w_ref, dis_ref, y_ref):
    xw = jnp.dot(x_ref[...], w_ref[...], preferred_element_type=jnp.float32)
    y_ref[...] = xw * dis_ref[...]


def _scaled_xw(x, w1, dis2):
    grid = 10
    blk = N // grid
    return pl.pallas_call(
        _mm_body,
        grid=(grid,),
        in_specs=[
            pl.BlockSpec((blk, C), lambda i: (i, 0)),
            pl.BlockSpec((C, C), lambda i: (0, 0)),
            pl.BlockSpec((blk, 1), lambda i: (i, 0)),
        ],
        out_specs=pl.BlockSpec((blk, C), lambda i: (i, 0)),
        out_shape=jax.ShapeDtypeStruct((N, C), jnp.float32),
    )(x, w1, dis2)


def _agg_body(edge_hbm, y_hbm, dis_hbm, zbig_hbm, zeros_hbm,
              agg_out, s_out,
              idx0, idx1, rows0, rows1, dis_v, s_hist, tmp, acc,
              agg_sh, s_sh,
              semi0, semi1, semg0, semg1):
    c = lax.axis_index("c")
    s = lax.axis_index("s")
    wid = s * NC + c
    idx = (idx0, idx1)
    rows = (rows0, rows1)
    semi = (semi0, semi1)
    semg = (semg0, semg1)
    nk = (NCHUNK - 1 - wid) // NW + 1

    def fireidx(k, b):
        ci = wid + k * NW
        pltpu.async_copy(edge_hbm.at[:, ci], idx[b], semi[b])

    def firegather(b):
        pltpu.make_async_copy(edge_hbm.at[:, 0], idx[b], semi[b]).wait()
        pltpu.async_copy(y_hbm.at[idx[b].at[0]], rows[b], semg[b])

    def waitgather(b):
        pltpu.make_async_copy(y_hbm.at[idx[b].at[0]], rows[b],
                              semg[b]).wait()

    def shist(b):
        for j in range(CH // VL):
            sl = pl.ds(j * VL, VL)
            dv = plsc.load_gather(dis_v, [idx[b][1, sl]])
            plsc.addupdate_scatter(s_hist, [idx[b][0, sl]], dv)

    def scatter(b):
        pltpu.sync_copy(rows[b], agg_sh.at[idx[b].at[1]], add=True)

    @pl.when(s < N // RB)
    def _():
        pltpu.sync_copy(zbig_hbm.at[pl.ds(s * RB, RB)],
                        agg_sh.at[pl.ds(s * RB, RB)])

    pltpu.sync_copy(dis_hbm, dis_v)
    pltpu.sync_copy(zeros_hbm, s_hist)

    fireidx(0, 0)
    fireidx(1, 1)
    firegather(0)
    plsc.subcore_barrier()

    @pl.loop(0, nk, step=2)
    def _(k):
        @pl.when(k + 1 < nk)
        def _():
            firegather(1)

        shist(0)
        waitgather(0)
        scatter(0)

        @pl.when(k + 2 < nk)
        def _():
            fireidx(k + 2, 0)
            firegather(0)

        @pl.when(k + 1 < nk)
        def _():
            shist(1)
            waitgather(1)
            scatter(1)

        @pl.when(k + 3 < nk)
        def _():
            fireidx(k + 3, 1)

    plsc.subcore_barrier()

    @pl.when(s < N // RB)
    def _():
        pltpu.sync_copy(agg_sh.at[pl.ds(s * RB, RB)],
                        agg_out.at[c, pl.ds(s * RB, RB)])

    _reduce_hists(s_hist, s_sh, tmp, acc, s_out, c, s)


def _aggregate(edge3, y, dis1, zbig, zeros_n):
    return pl.kernel(
        _agg_body,
        out_type=(
            jax.ShapeDtypeStruct((NC, N, C), jnp.float32),
            jax.ShapeDtypeStruct((NC * N,), jnp.float32),
        ),
        mesh=_mesh(),
        compiler_params=_SC_PARAMS,
        scratch_types=[
            pltpu.VMEM((2, CH), jnp.int32),
            pltpu.VMEM((2, CH), jnp.int32),
            pltpu.VMEM((CH, C), jnp.float32),
            pltpu.VMEM((CH, C), jnp.float32),
            pltpu.VMEM((N,), jnp.float32),
            pltpu.VMEM((N,), jnp.float32),
            pltpu.VMEM((SB,), jnp.float32),
            pltpu.VMEM((SB,), jnp.float32),
            pltpu.VMEM_SHARED((N, C), jnp.float32),
            pltpu.VMEM_SHARED((NS * N,), jnp.float32),
            pltpu.SemaphoreType.DMA,
            pltpu.SemaphoreType.DMA,
            pltpu.SemaphoreType.DMA,
            pltpu.SemaphoreType.DMA,
        ],
    )(edge3, y, dis1, zbig, zeros_n)


def _final_body(agg0_ref, agg1_ref, y_ref, dis_ref, s0_ref, s1_ref,
                b1_ref, w2_ref, b2_ref, out_ref, vacc):
    i = pl.program_id(0)
    ng = pl.num_programs(0)

    @pl.when(i == 0)
    def _():
        vacc[...] = jnp.zeros_like(vacc)

    dis = dis_ref[...]
    a = agg0_ref[...] + agg1_ref[...] + y_ref[...]
    h = jnp.maximum(a * dis + b1_ref[...], 0.0)
    cw = dis * (s0_ref[...] + s1_ref[...]) + dis * dis
    vacc[...] += jnp.sum(cw * h, axis=0, keepdims=True)

    @pl.when(i == ng - 1)
    def _():
        t = jnp.dot(vacc[...] * (1.0 / N), w2_ref[...],
                    preferred_element_type=jnp.float32) + b2_ref[...]
        m = jnp.max(t, axis=1, keepdims=True)
        lse = jnp.log(jnp.sum(jnp.exp(t - m), axis=1, keepdims=True)) + m
        out_ref[...] = t - lse


def _head(agg0, agg1, y, dis2, s0, s1, b1r, w2, b2r):
    grid = 10
    blk = N // grid
    return pl.pallas_call(
        _final_body,
        grid=(grid,),
        in_specs=[
            pl.BlockSpec((blk, C), lambda i: (i, 0)),
            pl.BlockSpec((blk, C), lambda i: (i, 0)),
            pl.BlockSpec((blk, C), lambda i: (i, 0)),
            pl.BlockSpec((blk, 1), lambda i: (i, 0)),
            pl.BlockSpec((blk, 1), lambda i: (i, 0)),
            pl.BlockSpec((blk, 1), lambda i: (i, 0)),
            pl.BlockSpec((1, C), lambda i: (0, 0)),
            pl.BlockSpec((C, C), lambda i: (0, 0)),
            pl.BlockSpec((1, C), lambda i: (0, 0)),
        ],
        out_specs=pl.BlockSpec((1, C), lambda i: (0, 0)),
        out_shape=jax.ShapeDtypeStruct((1, C), jnp.float32),
        scratch_shapes=[pltpu.VMEM((1, C), jnp.float32)],
    )(agg0, agg1, y, dis2, s0, s1, b1r, w2, b2r)


@jax.jit
def kernel(x, edge_index, Wf, bf, W1, b1, W2, b2):
    del Wf, bf
    dst_idx = edge_index[1]
    edge3 = edge_index.reshape(2, NCHUNK, CH)

    zbig = jnp.zeros((N, C), jnp.float32)
    zeros_n = jnp.zeros((N,), jnp.float32)

    deg_flat = _degrees(dst_idx, zeros_n)
    dis1 = lax.rsqrt(deg_flat[:N] + deg_flat[N:] + 1.0)
    dis2 = dis1.reshape(N, 1)

    y = _scaled_xw(x, W1, dis2)

    agg2, s_flat = _aggregate(edge3, y, dis1, zbig, zeros_n)

    return _head(agg2[0], agg2[1], y, dis2,
                 s_flat[:N].reshape(N, 1), s_flat[N:].reshape(N, 1),
                 b1.reshape(1, C), W2, b2.reshape(1, C))
